# Initial kernel scaffold; baseline (speedup 1.0000x reference)
import functools

import jax
import jax.numpy as jnp
from jax import lax
from jax.experimental import pallas as pl
from jax.experimental.pallas import tpu as pltpu
from jax.experimental.pallas import tpu_sc as plsc

_CHUNK = 128


@functools.lru_cache(maxsize=None)
def _make_gather(num_rows: int, vocab: int, d_model: int):
    info = plsc.get_sparse_core_info()
    nc, ns = info.num_cores, info.num_subcores
    nw = nc * ns
    assert num_rows % (nw * _CHUNK) == 0
    chunks_per_w = num_rows // (nw * _CHUNK)
    rows_per_w = chunks_per_w * _CHUNK

    mesh = plsc.VectorSubcoreMesh(core_axis_name="c", subcore_axis_name="s")

    @functools.partial(
        pl.kernel,
        mesh=mesh,
        out_type=jax.ShapeDtypeStruct((num_rows, d_model), jnp.float32),
        scratch_types=[
            pltpu.VMEM((chunks_per_w, _CHUNK), jnp.int32),
            pltpu.VMEM((_CHUNK, d_model), jnp.float32),
            pltpu.VMEM((_CHUNK, d_model), jnp.float32),
            pltpu.SemaphoreType.DMA,
            pltpu.SemaphoreType.DMA,
        ],
    )
    def emb(idx_hbm, table_hbm, out_hbm, idx_v, rows0, rows1, sem0, sem1):
        wid = lax.axis_index("s") * nc + lax.axis_index("c")
        base_chunk = wid * chunks_per_w
        out_base = wid * rows_per_w

        pltpu.sync_copy(idx_hbm.at[pl.ds(base_chunk, chunks_per_w)], idx_v)

        def gather(j, buf, sem):
            pltpu.async_copy(table_hbm.at[idx_v.at[j]], buf, sem)

        def write(j, buf):
            pltpu.sync_copy(buf, out_hbm.at[pl.ds(out_base + j * _CHUNK, _CHUNK)])

        gather(0, rows0, sem0)
        gather(1, rows1, sem1)

        def body(g, _):
            j0 = 2 * g
            pltpu.make_async_copy(table_hbm.at[idx_v.at[j0]], rows0, sem0).wait()
            write(j0, rows0)

            @pl.when(j0 + 2 < chunks_per_w)
            def _():
                gather(j0 + 2, rows0, sem0)

            pltpu.make_async_copy(table_hbm.at[idx_v.at[j0 + 1]], rows1, sem1).wait()
            write(j0 + 1, rows1)

            @pl.when(j0 + 3 < chunks_per_w)
            def _():
                gather(j0 + 3, rows1, sem1)

            return 0

        lax.fori_loop(0, chunks_per_w // 2, body, 0)

    return emb


def kernel(token_ids, embeddings):
    batch, hist = token_ids.shape
    vocab, d_model = embeddings.shape
    num_rows = batch * hist
    idx = token_ids.reshape(num_rows // _CHUNK, _CHUNK).astype(jnp.int32)
    out = _make_gather(num_rows, vocab, d_model)(idx, embeddings)
    return out.reshape(batch, hist, d_model)


# speedup vs baseline: 1.8398x; 1.8398x over previous
import functools

import jax
import jax.numpy as jnp
from jax import lax
from jax.experimental import pallas as pl
from jax.experimental.pallas import tpu as pltpu
from jax.experimental.pallas import tpu_sc as plsc

_CHUNK = 128


@functools.lru_cache(maxsize=None)
def _make_gather(num_rows: int, vocab: int, d_model: int):
    info = plsc.get_sparse_core_info()
    nc, ns = info.num_cores, info.num_subcores
    nw = nc * ns
    assert num_rows % (nw * _CHUNK) == 0
    chunks_per_w = num_rows // (nw * _CHUNK)
    rows_per_w = chunks_per_w * _CHUNK

    mesh = plsc.VectorSubcoreMesh(core_axis_name="c", subcore_axis_name="s")

    @functools.partial(
        pl.kernel,
        mesh=mesh,
        out_type=jax.ShapeDtypeStruct((num_rows, d_model), jnp.float32),
        compiler_params=pltpu.CompilerParams(use_tc_tiling_on_sc=False),
        scratch_types=[
            pltpu.VMEM((chunks_per_w, _CHUNK), jnp.int32),
            pltpu.VMEM((_CHUNK, d_model), jnp.float32),
            pltpu.VMEM((_CHUNK, d_model), jnp.float32),
            pltpu.SemaphoreType.DMA,
            pltpu.SemaphoreType.DMA,
        ],
    )
    def emb(idx_hbm, table_hbm, out_hbm, idx_v, rows0, rows1, sem0, sem1):
        wid = lax.axis_index("s") * nc + lax.axis_index("c")
        base_chunk = wid * chunks_per_w
        out_base = wid * rows_per_w

        pltpu.sync_copy(idx_hbm.at[pl.ds(base_chunk, chunks_per_w)], idx_v)

        def gather(j, buf, sem):
            pltpu.async_copy(table_hbm.at[idx_v.at[j]], buf, sem)

        def write(j, buf):
            pltpu.sync_copy(buf, out_hbm.at[pl.ds(out_base + j * _CHUNK, _CHUNK)])

        gather(0, rows0, sem0)
        gather(1, rows1, sem1)

        def body(g, _):
            j0 = 2 * g
            pltpu.make_async_copy(table_hbm.at[idx_v.at[j0]], rows0, sem0).wait()
            write(j0, rows0)

            @pl.when(j0 + 2 < chunks_per_w)
            def _():
                gather(j0 + 2, rows0, sem0)

            pltpu.make_async_copy(table_hbm.at[idx_v.at[j0 + 1]], rows1, sem1).wait()
            write(j0 + 1, rows1)

            @pl.when(j0 + 3 < chunks_per_w)
            def _():
                gather(j0 + 3, rows1, sem1)

            return 0

        lax.fori_loop(0, chunks_per_w // 2, body, 0)

    return emb


def kernel(token_ids, embeddings):
    batch, hist = token_ids.shape
    vocab, d_model = embeddings.shape
    num_rows = batch * hist
    idx = token_ids.reshape(num_rows // _CHUNK, _CHUNK).astype(jnp.int32)
    out = _make_gather(num_rows, vocab, d_model)(idx, embeddings)
    return out.reshape(batch, hist, d_model)


# speedup vs baseline: 2.3922x; 1.3002x over previous
import functools

import jax
import jax.numpy as jnp
from jax import lax
from jax.experimental import pallas as pl
from jax.experimental.pallas import tpu as pltpu
from jax.experimental.pallas import tpu_sc as plsc

_HP = 56
_DP = 128
_HPAD = 52


@functools.lru_cache(maxsize=None)
def _make_gather(batch: int, hist: int, vocab: int, d_model: int):
    info = plsc.get_sparse_core_info()
    nc, ns = info.num_cores, info.num_subcores
    nw = nc * ns
    assert batch % (2 * nw) == 0
    b_per_w = batch // nw
    chunk = 2 * _HPAD
    n_chunks = b_per_w // 2

    mesh = plsc.VectorSubcoreMesh(core_axis_name="c", subcore_axis_name="s")

    @functools.partial(
        pl.kernel,
        mesh=mesh,
        out_type=jax.ShapeDtypeStruct((batch, _HP, _DP), jnp.float32),
        compiler_params=pltpu.CompilerParams(use_tc_tiling_on_sc=False),
        scratch_types=[
            pltpu.VMEM((n_chunks, chunk), jnp.int32),
            pltpu.VMEM((chunk, d_model), jnp.float32),
            pltpu.VMEM((chunk, d_model), jnp.float32),
            pltpu.SemaphoreType.DMA,
            pltpu.SemaphoreType.DMA,
        ],
    )
    def emb(idx_hbm, table_hbm, out_hbm, idx_v, rows0, rows1, sem0, sem1):
        wid = lax.axis_index("s") * nc + lax.axis_index("c")
        b_base = wid * b_per_w

        pltpu.sync_copy(idx_hbm.at[pl.ds(wid * n_chunks, n_chunks)], idx_v)

        def gather(k, buf, sem):
            pltpu.async_copy(table_hbm.at[idx_v.at[k]], buf, sem)

        def write(k, buf):
            b = b_base + 2 * k
            pltpu.sync_copy(buf.at[pl.ds(0, hist)],
                            out_hbm.at[b, pl.ds(0, hist), pl.ds(0, d_model)])
            pltpu.sync_copy(buf.at[pl.ds(_HPAD, hist)],
                            out_hbm.at[b + 1, pl.ds(0, hist), pl.ds(0, d_model)])

        gather(0, rows0, sem0)
        gather(1, rows1, sem1)

        def body(g, _):
            k0 = 2 * g
            pltpu.make_async_copy(table_hbm.at[idx_v.at[k0]], rows0, sem0).wait()
            write(k0, rows0)

            @pl.when(k0 + 2 < n_chunks)
            def _():
                gather(k0 + 2, rows0, sem0)

            pltpu.make_async_copy(table_hbm.at[idx_v.at[k0 + 1]], rows1, sem1).wait()
            write(k0 + 1, rows1)

            @pl.when(k0 + 3 < n_chunks)
            def _():
                gather(k0 + 3, rows1, sem1)

            return 0

        lax.fori_loop(0, n_chunks // 2, body, 0)

    return emb


def kernel(token_ids, embeddings):
    batch, hist = token_ids.shape
    vocab, d_model = embeddings.shape
    idx = token_ids.astype(jnp.int32)
    fill = (jnp.arange(batch, dtype=jnp.int32)[:, None] * 2
            + jnp.arange(2, dtype=jnp.int32)[None, :]) % vocab
    idxp = jnp.concatenate([idx, fill], axis=1)
    idxp = idxp.reshape(batch // 2, 2 * _HPAD)
    outp = _make_gather(batch, hist, vocab, d_model)(idxp, embeddings)
    return outp[:, :hist, :d_model]


# speedup vs baseline: 2.5199x; 1.0534x over previous
import functools

import jax
import jax.numpy as jnp
from jax import lax
from jax.experimental import pallas as pl
from jax.experimental.pallas import tpu as pltpu
from jax.experimental.pallas import tpu_sc as plsc

_HP = 56
_DP = 128
_HPAD = 52


@functools.lru_cache(maxsize=None)
def _make_gather(batch: int, hist: int, vocab: int, d_model: int):
    info = plsc.get_sparse_core_info()
    nc, ns = info.num_cores, info.num_subcores
    nw = nc * ns
    assert batch % (2 * nw) == 0
    b_per_w = batch // nw
    chunk = 2 * _HPAD
    n_chunks = b_per_w // 2

    mesh = plsc.VectorSubcoreMesh(core_axis_name="c", subcore_axis_name="s")

    @functools.partial(
        pl.kernel,
        mesh=mesh,
        out_type=jax.ShapeDtypeStruct((batch, _HP, _DP), jnp.float32),
        compiler_params=pltpu.CompilerParams(use_tc_tiling_on_sc=False),
        scratch_types=[
            pltpu.VMEM((n_chunks, chunk), jnp.int32),
            pltpu.VMEM((chunk, d_model), jnp.float32),
            pltpu.VMEM((chunk, d_model), jnp.float32),
            pltpu.VMEM((chunk, d_model), jnp.float32),
            pltpu.VMEM((chunk, d_model), jnp.float32),
            pltpu.SemaphoreType.DMA,
            pltpu.SemaphoreType.DMA,
            pltpu.SemaphoreType.DMA,
            pltpu.SemaphoreType.DMA,
        ],
    )
    def emb(idx_hbm, table_hbm, out_hbm, idx_v,
            rows0, rows1, rows2, rows3, sem0, sem1, sem2, sem3):
        wid = lax.axis_index("s") * nc + lax.axis_index("c")
        b_base = wid * b_per_w
        bufs = (rows0, rows1, rows2, rows3)
        sems = (sem0, sem1, sem2, sem3)
        nbuf = len(bufs)

        pltpu.sync_copy(idx_hbm.at[pl.ds(wid * n_chunks, n_chunks)], idx_v)

        def gather(k, buf, sem):
            pltpu.async_copy(table_hbm.at[idx_v.at[k]], buf, sem)

        def write(k, buf):
            b = b_base + 2 * k
            pltpu.sync_copy(buf.at[pl.ds(0, hist)],
                            out_hbm.at[b, pl.ds(0, hist), pl.ds(0, d_model)])
            pltpu.sync_copy(buf.at[pl.ds(_HPAD, hist)],
                            out_hbm.at[b + 1, pl.ds(0, hist), pl.ds(0, d_model)])

        for i in range(nbuf):
            gather(i, bufs[i], sems[i])

        def body(g, _):
            k0 = nbuf * g
            for i in range(nbuf):
                k = k0 + i
                pltpu.make_async_copy(
                    table_hbm.at[idx_v.at[k]], bufs[i], sems[i]).wait()
                write(k, bufs[i])

                @pl.when(k + nbuf < n_chunks)
                def _():
                    gather(k + nbuf, bufs[i], sems[i])

            return 0

        lax.fori_loop(0, n_chunks // nbuf, body, 0)

    return emb


def kernel(token_ids, embeddings):
    batch, hist = token_ids.shape
    vocab, d_model = embeddings.shape
    idx = token_ids.astype(jnp.int32)
    fill = (jnp.arange(batch, dtype=jnp.int32)[:, None] * 2
            + jnp.arange(2, dtype=jnp.int32)[None, :]) % vocab
    idxp = jnp.concatenate([idx, fill], axis=1)
    idxp = idxp.reshape(batch // 2, 2 * _HPAD)
    outp = _make_gather(batch, hist, vocab, d_model)(idxp, embeddings)
    return outp[:, :hist, :d_model]
